# baseline (speedup 1.0000x reference)
import functools

import jax
import jax.numpy as jnp
from jax import lax
from jax.experimental import pallas as pl

_Q = 4


def _body(*refs, n, c, p, bn):
    pcl_refs = refs[:_Q]
    lab_ref, w_ref, pclab_ref, pcp_ref, pcc_ref, imw_ref, im_ref = refs[_Q:-1]
    out_ref = refs[-1]
    i = pl.program_id(0)
    stripe = _Q * bn
    im_r = im_ref[...].reshape(1, c)
    gate = (im_r[:, 0:1] != 0.0).astype(jnp.float32)

    bg_part = jnp.zeros((1, 1), jnp.float32)
    for q in range(_Q):
        x = pcl_refs[q][:, 0:1]
        z = jnp.log(jnp.where(x > 0.0, x, 1.0))
        base = i * stripe + q * bn
        valid1 = base + lax.broadcasted_iota(jnp.int32, (bn,), 0) < n
        lab = lab_ref[pl.ds(q * bn, bn)]
        w = w_ref[pl.ds(q * bn, bn)]
        wm = jnp.where(valid1 & (lab == 0), w, 0.0)
        bg_part += lax.dot_general(
            wm.reshape(1, bn), z,
            dimension_numbers=(((1,), (0,)), ((), ())),
            preferred_element_type=jnp.float32)

    @pl.when(i == 0)
    def _():
        pclab = pclab_ref[...].reshape(1, p)
        imnz = (im_r != 0.0).astype(jnp.float32)
        onehot = (lax.broadcasted_iota(jnp.int32, (c, p), 0)
                  == pclab).astype(jnp.float32)
        im_at_nz = lax.dot_general(
            imnz, onehot,
            dimension_numbers=(((1,), (0,)), ((), ())),
            preferred_element_type=jnp.float32)
        pcp = pcp_ref[...].reshape(1, p)
        fg_mask = ((im_at_nz > 0.5) & (pclab > 0)
                   & (pcc_ref[...].reshape(1, p) > 0.0))
        fg = jnp.sum(
            jnp.where(fg_mask,
                      imw_ref[...].reshape(1, p) * jnp.log(pcp), 0.0),
            keepdims=True)
        out_ref[...] = fg * jnp.float32(-1.0 / n)

    out_ref[...] += (gate * bg_part) * jnp.float32(-1.0 / n)


@functools.partial(jax.jit, static_argnames=("n", "c", "p", "bn"))
def _loss(pcl_prob, labels, w, pc_labels, pc_probs, pc_count, img_w,
          im_labels, *, n, c, p, bn):
    stripe = _Q * bn
    grid = -(-n // stripe)
    full = lambda i: (0,)

    def pcl_spec(q):
        return pl.BlockSpec((bn, c), lambda i, q=q: (i * _Q + q, 0))

    out = pl.pallas_call(
        functools.partial(_body, n=n, c=c, p=p, bn=bn),
        grid=(grid,),
        in_specs=[pcl_spec(q) for q in range(_Q)] + [
            pl.BlockSpec((stripe,), lambda i: (i,)),
            pl.BlockSpec((stripe,), lambda i: (i,)),
            pl.BlockSpec((p,), full),
            pl.BlockSpec((p,), full),
            pl.BlockSpec((p,), full),
            pl.BlockSpec((p,), full),
            pl.BlockSpec((c,), full),
        ],
        out_specs=pl.BlockSpec((1, 1), lambda i: (0, 0)),
        out_shape=jax.ShapeDtypeStruct((1, 1), jnp.float32),
    )(*([pcl_prob] * _Q), labels, w, pc_labels, pc_probs, pc_count, img_w,
      im_labels)
    return out[0, 0]


def kernel(pcl_prob, labels, cls_loss_weights, gt_assignment, pc_labels,
           pc_probs, pc_count, img_cls_loss_weights, im_labels_real):
    n, c = pcl_prob.shape
    p = pc_labels.shape[0]
    return _loss(pcl_prob, labels, cls_loss_weights, pc_labels, pc_probs,
                 pc_count, img_cls_loss_weights, im_labels_real,
                 n=n, c=c, p=p, bn=1024)
